# baseline (speedup 1.0000x reference)
import jax
import jax.numpy as jnp
from jax.experimental import pallas as pl
from jax.experimental.pallas import tpu as pltpu

_IN = 6
_HID = 32
_OUT = 2
_PACK = 64
_ROW_IN = _PACK * _IN
_ROW_OUT = _PACK * _OUT
_TILE_R = 512


def _mlp_kernel(x_ref, w1_ref, w2_ref, b_ref, o_ref):
    h = jnp.dot(x_ref[...], w1_ref[...],
                preferred_element_type=jnp.float32)
    t = jnp.tanh(h + b_ref[0:1, :])
    u = jnp.tanh(jnp.dot(t, w2_ref[...],
                         preferred_element_type=jnp.float32)
                 + b_ref[1:2, 0:_ROW_OUT])
    o_ref[...] = 0.5 * u + 0.5


def kernel(x_batch, param_slab):
    B = x_batch.shape[0]
    span = _PACK * _TILE_R
    b_pad = pl.cdiv(B, span) * span
    if b_pad != B:
        x_batch = jnp.pad(x_batch.astype(jnp.float32),
                          ((0, b_pad - B), (0, 0)))
    rows = b_pad // _PACK
    xp = x_batch.reshape(rows, _ROW_IN)

    w1 = param_slab[0:_IN, 0:_HID]
    b1 = param_slab[8, 0:_HID]
    w2 = param_slab[9:9 + _OUT, 0:_HID].T
    b2 = param_slab[11, 0:_OUT]
    eye = jnp.eye(_PACK, dtype=jnp.float32)
    w1d = jnp.kron(eye, 0.5 * w1)
    w2d = jnp.kron(eye, 0.25 * w2)
    b2f = 0.5 * (b2 + 0.5 * jnp.sum(w2, axis=0))
    bb = jnp.zeros((8, _PACK * _HID), jnp.float32)
    bb = bb.at[0, :].set(jnp.tile(0.5 * b1, _PACK))
    bb = bb.at[1, 0:_ROW_OUT].set(jnp.tile(b2f, _PACK))

    n_tiles = rows // _TILE_R
    out = pl.pallas_call(
        _mlp_kernel,
        out_shape=jax.ShapeDtypeStruct((rows, _ROW_OUT), jnp.float32),
        grid=(n_tiles,),
        in_specs=[
            pl.BlockSpec((_TILE_R, _ROW_IN), lambda i: (i, 0)),
            pl.BlockSpec((_ROW_IN, _PACK * _HID), lambda i: (0, 0)),
            pl.BlockSpec((_PACK * _HID, _ROW_OUT), lambda i: (0, 0)),
            pl.BlockSpec((8, _PACK * _HID), lambda i: (0, 0)),
        ],
        out_specs=pl.BlockSpec((_TILE_R, _ROW_OUT), lambda i: (i, 0)),
        compiler_params=pltpu.CompilerParams(
            dimension_semantics=("parallel",)),
    )(xp, w1d, w2d, bb)
    return out.reshape(b_pad, _OUT)[:B]


# speedup vs baseline: 13.1541x; 13.1541x over previous
import jax
import jax.numpy as jnp
from jax.experimental import pallas as pl
from jax.experimental.pallas import tpu as pltpu

_IN = 6
_HID = 32
_OUT = 2
_TILE_L = 32768


def _mlp_kernel(x_ref, p_ref, o_ref):
    h = jnp.dot(p_ref[0:_HID, 0:8], x_ref[...],
                preferred_element_type=jnp.float32)
    t = jnp.tanh(h + p_ref[0:_HID, 8:9])
    o = jnp.dot(p_ref[_HID:_HID + 8, 0:_HID], t,
                preferred_element_type=jnp.float32)
    u = jnp.tanh(o + p_ref[_HID:_HID + 8, 33:34])
    o_ref[...] = 0.5 * u + 0.5


def kernel(x_batch, param_slab):
    B = x_batch.shape[0]
    b_pad = pl.cdiv(B, _TILE_L) * _TILE_L
    if b_pad != B:
        x_batch = jnp.pad(x_batch.astype(jnp.float32),
                          ((0, b_pad - B), (0, 0)))
    xt = jnp.pad(x_batch, ((0, 0), (0, 2))).T

    w1 = param_slab[0:_IN, 0:_HID]
    b1 = param_slab[8, 0:_HID]
    w2 = param_slab[9:9 + _OUT, 0:_HID].T
    b2 = param_slab[11, 0:_OUT]
    b2f = 0.5 * (b2 + 0.5 * jnp.sum(w2, axis=0))
    p = jnp.zeros((40, 128), jnp.float32)
    p = p.at[0:_HID, 0:_IN].set(0.5 * w1.T)
    p = p.at[0:_HID, 8].set(0.5 * b1)
    p = p.at[_HID:_HID + _OUT, 0:_HID].set(0.25 * w2.T)
    p = p.at[_HID:_HID + _OUT, 33].set(b2f)

    n_tiles = b_pad // _TILE_L
    out = pl.pallas_call(
        _mlp_kernel,
        out_shape=jax.ShapeDtypeStruct((8, b_pad), jnp.float32),
        grid=(n_tiles,),
        in_specs=[
            pl.BlockSpec((8, _TILE_L), lambda i: (0, i)),
            pl.BlockSpec((40, 128), lambda i: (0, 0)),
        ],
        out_specs=pl.BlockSpec((8, _TILE_L), lambda i: (0, i)),
        compiler_params=pltpu.CompilerParams(
            dimension_semantics=("parallel",)),
    )(xt, p)
    return out[0:_OUT, :B].T


# speedup vs baseline: 18.9395x; 1.4398x over previous
import jax
import jax.numpy as jnp
from jax.experimental import pallas as pl
from jax.experimental.pallas import tpu as pltpu

_IN = 6
_HID = 32
_OUT = 2
_TILE_L = 32768


def _mlp_kernel(x_ref, s_ref, o_ref):
    w1t = s_ref[0:_IN, 0:_HID] * 0.5
    h = jax.lax.dot_general(w1t, x_ref[0:_IN, :],
                            (((0,), (0,)), ((), ())),
                            preferred_element_type=jnp.float32)
    b1c = jnp.transpose(s_ref[8:9, 0:_HID]) * 0.5
    t = jnp.tanh(h + b1c)
    w2t = s_ref[9:9 + _OUT, 0:_HID]
    o = jnp.dot(w2t * 0.25, t,
                preferred_element_type=jnp.float32)
    b2c = (jnp.transpose(s_ref[11:12, 0:_OUT])
           + 0.5 * jnp.sum(w2t, axis=1, keepdims=True)) * 0.5
    o_ref[...] = 0.5 * jnp.tanh(o + b2c) + 0.5


def kernel(x_batch, param_slab):
    B = x_batch.shape[0]
    b_pad = pl.cdiv(B, _TILE_L) * _TILE_L
    if b_pad != B:
        x_batch = jnp.pad(x_batch.astype(jnp.float32),
                          ((0, b_pad - B), (0, 0)))
    xt = jnp.pad(x_batch, ((0, 0), (0, 2))).T

    n_tiles = b_pad // _TILE_L
    out = pl.pallas_call(
        _mlp_kernel,
        out_shape=jax.ShapeDtypeStruct((_OUT, b_pad), jnp.float32),
        grid=(n_tiles,),
        in_specs=[
            pl.BlockSpec((8, _TILE_L), lambda i: (0, i)),
            pl.BlockSpec((16, 128), lambda i: (0, 0)),
        ],
        out_specs=pl.BlockSpec((_OUT, _TILE_L), lambda i: (0, i)),
        compiler_params=pltpu.CompilerParams(
            dimension_semantics=("parallel",)),
    )(xt, param_slab)
    return out[:, :B].T


# speedup vs baseline: 36.4309x; 1.9235x over previous
import jax
import jax.numpy as jnp
from jax.experimental import pallas as pl
from jax.experimental.pallas import tpu as pltpu

_IN = 6
_HID = 32
_OUT = 2
_TILE_L = 65536


def _mlp_kernel(x_ref, s_ref, o_ref):
    w1t = s_ref[0:_IN, 0:_HID] * 0.5
    h = jax.lax.dot_general(w1t, x_ref[...],
                            (((0,), (0,)), ((), ())),
                            preferred_element_type=jnp.float32)
    b1c = jnp.transpose(s_ref[8:9, 0:_HID]) * 0.5
    t = jnp.tanh(h + b1c)
    w2t = s_ref[9:9 + _OUT, 0:_HID]
    o = jnp.dot(w2t * 0.25, t,
                preferred_element_type=jnp.float32)
    b2c = (jnp.transpose(s_ref[11:12, 0:_OUT])
           + 0.5 * jnp.sum(w2t, axis=1, keepdims=True)) * 0.5
    o_ref[...] = 0.5 * jnp.tanh(o + b2c) + 0.5


def kernel(x_batch, param_slab):
    B = x_batch.shape[0]
    b_pad = pl.cdiv(B, _TILE_L) * _TILE_L
    if b_pad != B:
        x_batch = jnp.pad(x_batch.astype(jnp.float32),
                          ((0, b_pad - B), (0, 0)))
    xt = x_batch.T

    n_tiles = b_pad // _TILE_L
    out = pl.pallas_call(
        _mlp_kernel,
        out_shape=jax.ShapeDtypeStruct((_OUT, b_pad), jnp.float32),
        grid=(n_tiles,),
        in_specs=[
            pl.BlockSpec((_IN, _TILE_L), lambda i: (0, i)),
            pl.BlockSpec((16, 128), lambda i: (0, 0)),
        ],
        out_specs=pl.BlockSpec((_OUT, _TILE_L), lambda i: (0, i)),
        compiler_params=pltpu.CompilerParams(
            dimension_semantics=("parallel",)),
    )(xt, param_slab)
    return out[:, :B].T
